# baseline (speedup 1.0000x reference)
import functools

import jax
import jax.numpy as jnp
from jax import lax
from jax.experimental import pallas as pl
from jax.experimental.pallas import tpu as pltpu
from jax.experimental.pallas import tpu_sc as plsc

D = 128
KCH = 3
SG = 128
CB = 256
TB = 256
NW = 32
LANE = 128
G_OUT = 25000


@functools.lru_cache(maxsize=None)
def _sc_gather(n_rows, n_feat, m_in, m_pad, nch):
    rc = KCH * CB
    nsub = rc // LANE
    mesh = plsc.VectorSubcoreMesh(core_axis_name="c", subcore_axis_name="s")

    @functools.partial(
        pl.kernel,
        mesh=mesh,
        out_type=jax.ShapeDtypeStruct((m_pad * KCH, n_feat), jnp.float32),
        scratch_types=[
            pltpu.VMEM((rc,), jnp.int32),
            pltpu.VMEM((nsub, LANE), jnp.int32),
            pltpu.VMEM((nsub, LANE), jnp.int32),
            pltpu.VMEM((rc, n_feat), jnp.float32),
            pltpu.SemaphoreType.DMA,
        ],
    )
    def gather_kernel(x_hbm, imap_hbm, om_flat_hbm, da_hbm,
                      om_v, idx_v, ci_v, rows_v, sem):
        wid = lax.axis_index("s") * 2 + lax.axis_index("c")

        def chunk(j, carry):
            cid = wid * nch + j
            pltpu.sync_copy(om_flat_hbm.at[pl.ds(cid * rc, rc)], om_v)
            for t in range(nsub):
                for i in range(LANE // 16):
                    idx_v[t, pl.ds(i * 16, 16)] = jnp.maximum(
                        om_v[pl.ds(t * LANE + i * 16, 16)], 0)
            cps = [pltpu.async_copy(imap_hbm.at[idx_v.at[t]], ci_v.at[t], sem)
                   for t in range(nsub)]
            for c in cps:
                c.wait()
            cps = [pltpu.async_copy(x_hbm.at[ci_v.at[t]],
                                    rows_v.at[pl.ds(t * LANE, LANE)], sem)
                   for t in range(nsub)]
            for c in cps:
                c.wait()
            pltpu.sync_copy(rows_v, da_hbm.at[pl.ds(cid * rc, rc)])
            return carry

        lax.fori_loop(0, nch, chunk, 0)

    return gather_kernel


def _wprep_body(ws_ref, wsi_ref, wz_ref, wzi_ref, w2_ref, w3_ref):
    for k in range(KCH):
        w2_ref[k] = lax.dot_general(ws_ref[k], wsi_ref[...],
                                    (((0,), (1,)), ((), ())),
                                    preferred_element_type=jnp.float32)
        w3_ref[k] = lax.dot_general(wz_ref[k], wzi_ref[...],
                                    (((0,), (1,)), ((), ())),
                                    preferred_element_type=jnp.float32)


def _weight_prep(Ws, Ws_int, Wz, Wz_int):
    full = lambda s: pl.BlockSpec(s, lambda: tuple(0 for _ in s))
    return pl.pallas_call(
        _wprep_body,
        in_specs=[full((KCH, D, D)), full((D, D)), full((KCH, D, D)), full((D, D))],
        out_specs=[full((KCH, D, D)), full((KCH, D, D))],
        out_shape=[jax.ShapeDtypeStruct((KCH, D, D), jnp.float32),
                   jax.ShapeDtypeStruct((KCH, D, D), jnp.float32)],
    )(Ws, Ws_int, Wz, Wz_int)


def _main_body(rs_ref, da_hbm, om0_hbm, om1_hbm, om2_hbm, pool_hbm,
               trow_hbm, tcol_hbm,
               w2_ref, w3_ref, wp_ref, wpf_ref, out_ref,
               da_buf, om0_buf, om1_buf, om2_buf, pool_buf, trow_buf, tcol_buf,
               psum, tm2, sems):
    i = pl.program_id(0)
    rs = (rs_ref[i] // TB) * TB
    re = rs_ref[i + 1]
    nch = (re - rs + TB - 1) // TB
    seg0 = i * SG

    psum[...] = jnp.zeros((SG, D), jnp.float32)
    tm2[...] = jnp.zeros((SG, 1), jnp.int32)

    def chunk(j, carry):
        r0 = rs + j * TB
        cps = [
            pltpu.make_async_copy(da_hbm.at[pl.ds(r0, TB)], da_buf, sems.at[0]),
            pltpu.make_async_copy(om0_hbm.at[pl.ds(r0, TB)], om0_buf, sems.at[1]),
            pltpu.make_async_copy(om1_hbm.at[pl.ds(r0, TB)], om1_buf, sems.at[2]),
            pltpu.make_async_copy(om2_hbm.at[pl.ds(r0, TB)], om2_buf, sems.at[3]),
            pltpu.make_async_copy(pool_hbm.at[pl.ds(r0, TB)], pool_buf, sems.at[4]),
            pltpu.make_async_copy(trow_hbm.at[pl.ds(r0, TB)], trow_buf, sems.at[5]),
            pltpu.make_async_copy(tcol_hbm.at[pl.ds(r0, TB)], tcol_buf, sems.at[6]),
        ]
        for c in cps:
            c.start()
        for c in cps:
            c.wait()

        da = da_buf[...]
        d0 = jnp.where(om0_buf[...] >= 0, da[:, 0, :], 0.0)
        d1 = jnp.where(om1_buf[...] >= 0, da[:, 1, :], 0.0)
        d2 = jnp.where(om2_buf[...] >= 0, da[:, 2, :], 0.0)

        mm = lambda a, w: lax.dot_general(a, w, (((1,), (0,)), ((), ())),
                                          preferred_element_type=jnp.float32)
        y2 = mm(d0, w2_ref[0]) + mm(d1, w2_ref[1]) + mm(d2, w2_ref[2])
        y3 = mm(d0, w3_ref[0]) + mm(d1, w3_ref[1]) + mm(d2, w3_ref[2])
        y1 = lax.dot_general(d0, wp_ref[...], (((1,), (1,)), ((), ())),
                             preferred_element_type=jnp.float32)

        tc = tcol_buf[...]
        tmp = d0
        tmp = jnp.where(tc == 3, y3, tmp)
        tmp = jnp.where(tc == 2, y2, tmp)
        tmp = jnp.where(tc == 1, y1, tmp)

        rel = pool_buf[...] - seg0
        ohT = lax.broadcasted_iota(jnp.int32, (SG, TB), 0) == rel[None, :]
        psum[...] += lax.dot_general(ohT.astype(jnp.float32), tmp,
                                     (((1,), (0,)), ((), ())),
                                     preferred_element_type=jnp.float32)
        tm2[...] = jnp.maximum(
            tm2[...],
            jnp.max(jnp.where(ohT, trow_buf[...][None, :], 0),
                    axis=1, keepdims=True))
        return carry

    lax.fori_loop(0, nch, chunk, 0)

    p = psum[...]
    tm = tm2[...]
    act = jnp.where(p > 0, p, jnp.exp(p) - 1.0)
    act = jnp.where(tm != 0, act, p)
    fin = lax.dot_general(act, wpf_ref[...], (((1,), (1,)), ((), ())),
                          preferred_element_type=jnp.float32)
    out_ref[...] = jnp.where(tm == 1, fin, act)


@functools.lru_cache(maxsize=None)
def _main_call(m_pad, n_g):
    any_spec = pl.BlockSpec(memory_space=pltpu.MemorySpace.HBM)
    fullw = lambda s: pl.BlockSpec(s, lambda i: tuple(0 for _ in s))
    return pl.pallas_call(
        _main_body,
        grid=(n_g,),
        in_specs=[
            pl.BlockSpec(memory_space=pltpu.SMEM),
            any_spec,
            any_spec,
            any_spec,
            any_spec,
            any_spec,
            any_spec,
            any_spec,
            fullw((KCH, D, D)),
            fullw((KCH, D, D)),
            fullw((D, D)),
            fullw((D, D)),
        ],
        out_specs=pl.BlockSpec((SG, D), lambda i: (i, 0)),
        out_shape=jax.ShapeDtypeStruct((G_OUT, D), jnp.float32),
        scratch_shapes=[
            pltpu.VMEM((TB, KCH, D), jnp.float32),
            pltpu.VMEM((TB, 1), jnp.int32),
            pltpu.VMEM((TB, 1), jnp.int32),
            pltpu.VMEM((TB, 1), jnp.int32),
            pltpu.VMEM((TB,), jnp.int32),
            pltpu.VMEM((TB,), jnp.int32),
            pltpu.VMEM((TB, 1), jnp.int32),
            pltpu.VMEM((SG, D), jnp.float32),
            pltpu.VMEM((SG, 1), jnp.int32),
            pltpu.SemaphoreType.DMA((7,)),
        ],
    )


def kernel(x, initial_map, order_matrix, type_mask, pooling,
           Ws, Ws_int, Wz, Wz_int, Wp, Wpf):
    n, d = x.shape
    k_ch, m = order_matrix.shape
    m_in = initial_map.shape[0]
    nch_sc = -(-m // (NW * CB))
    m_pad = NW * CB * nch_sc
    n_g = -(-G_OUT // SG)

    imap = initial_map.astype(jnp.int32)
    om = order_matrix.astype(jnp.int32)
    om_t = jnp.pad(om.T, ((0, m_pad - m), (0, 0)), constant_values=-1)
    om_flat = om_t.reshape(m_pad * KCH)
    om_pad = jnp.pad(om, ((0, 0), (0, m_pad - m)), constant_values=-1)
    om0 = om_pad[0].reshape(m_pad, 1)
    om1 = om_pad[1].reshape(m_pad, 1)
    om2 = om_pad[2].reshape(m_pad, 1)
    pool_i = pooling.astype(jnp.int32)
    pool_pad = jnp.pad(pool_i, (0, m_pad - m), constant_values=jnp.int32(2 ** 30))
    type_pad = jnp.pad(type_mask.astype(jnp.int32), (0, m_pad - m))
    row_start = jnp.searchsorted(
        pool_i, (jnp.arange(n_g + 1) * SG).astype(jnp.int32)).astype(jnp.int32)

    da_flat = _sc_gather(n, d, m_in, m_pad, nch_sc)(x, imap, om_flat)
    da = da_flat.reshape(m_pad, KCH, d)
    w2, w3 = _weight_prep(Ws, Ws_int, Wz, Wz_int)
    return _main_call(m_pad, n_g)(
        row_start, da, om0, om1, om2, pool_pad, type_pad,
        type_pad.reshape(m_pad, 1), w2, w3, Wp, Wpf)


# speedup vs baseline: 1.3242x; 1.3242x over previous
import functools

import jax
import jax.numpy as jnp
from jax import lax
from jax.experimental import pallas as pl
from jax.experimental.pallas import tpu as pltpu
from jax.experimental.pallas import tpu_sc as plsc

D = 128
KCH = 3
SG = 128
CB = 128
TB = 256
NW = 32
LANE = 128
G_OUT = 25000


@functools.lru_cache(maxsize=None)
def _sc_gather(n_rows, n_feat, m_in, m_pad, nch):
    cbr = KCH * CB
    wrows = nch * cbr
    nidx = wrows // LANE
    npair = nch // 2
    mesh = plsc.VectorSubcoreMesh(core_axis_name="c", subcore_axis_name="s")

    @functools.partial(
        pl.kernel,
        mesh=mesh,
        out_type=jax.ShapeDtypeStruct((m_pad * KCH, n_feat), jnp.float32),
        scratch_types=[
            pltpu.VMEM((wrows,), jnp.int32),
            pltpu.VMEM((wrows,), jnp.int32),
            pltpu.VMEM((wrows,), jnp.int32),
            pltpu.VMEM((cbr, n_feat), jnp.float32),
            pltpu.VMEM((cbr, n_feat), jnp.float32),
            pltpu.SemaphoreType.DMA,
            pltpu.SemaphoreType.DMA,
            pltpu.SemaphoreType.DMA,
            pltpu.SemaphoreType.DMA,
            pltpu.SemaphoreType.DMA,
        ],
    )
    def gather_kernel(x_hbm, imap_hbm, om_flat_hbm, da_hbm,
                      om_all, idx_all, ci_all, buf_a, buf_b,
                      sem_i, sem_ga, sem_gb, sem_oa, sem_ob):
        wid = lax.axis_index("s") * 2 + lax.axis_index("c")
        base = wid * wrows

        pltpu.sync_copy(om_flat_hbm.at[pl.ds(base, wrows)], om_all)

        def clamp(i, carry):
            sl = pl.ds(i * 16, 16)
            idx_all[sl] = jnp.maximum(om_all[sl], 0)
            return carry

        lax.fori_loop(0, wrows // 16, clamp, 0)

        def fire_imap(i, carry):
            sl = pl.ds(i * LANE, LANE)
            pltpu.async_copy(imap_hbm.at[idx_all.at[sl]], ci_all.at[sl], sem_i)
            return carry

        lax.fori_loop(0, nidx, fire_imap, 0)
        pltpu.make_async_copy(om_flat_hbm.at[pl.ds(0, wrows)], ci_all,
                              sem_i).wait()

        def fire(cid, buf, sem):
            for t in range(cbr // LANE):
                off = cid * cbr + t * LANE
                pltpu.async_copy(x_hbm.at[ci_all.at[pl.ds(off, LANE)]],
                                 buf.at[pl.ds(t * LANE, LANE)], sem)

        def drain_g(buf, sem):
            pltpu.make_async_copy(x_hbm.at[pl.ds(0, cbr)], buf, sem).wait()

        def out_start(cid, buf, sem):
            pltpu.async_copy(buf, da_hbm.at[pl.ds(base + cid * cbr, cbr)], sem)

        def out_wait(cid, buf, sem):
            pltpu.make_async_copy(buf, da_hbm.at[pl.ds(base + cid * cbr, cbr)],
                                  sem).wait()

        fire(0, buf_a, sem_ga)

        def pair(jj, carry):
            j = jj * 2

            @pl.when(jj > 0)
            def _():
                out_wait(j - 1, buf_b, sem_ob)

            fire(j + 1, buf_b, sem_gb)
            drain_g(buf_a, sem_ga)
            out_start(j, buf_a, sem_oa)

            @pl.when(jj < npair - 1)
            def _():
                out_wait(j, buf_a, sem_oa)
                fire(j + 2, buf_a, sem_ga)

            drain_g(buf_b, sem_gb)
            out_start(j + 1, buf_b, sem_ob)
            return carry

        lax.fori_loop(0, npair, pair, 0)
        out_wait(nch - 2, buf_a, sem_oa)
        out_wait(nch - 1, buf_b, sem_ob)

    return gather_kernel


def _wprep_body(ws_ref, wsi_ref, wz_ref, wzi_ref, w23_ref):
    for k in range(KCH):
        w23_ref[k, :, 0:D] = lax.dot_general(
            ws_ref[k], wsi_ref[...], (((0,), (1,)), ((), ())),
            preferred_element_type=jnp.float32)
        w23_ref[k, :, D:2 * D] = lax.dot_general(
            wz_ref[k], wzi_ref[...], (((0,), (1,)), ((), ())),
            preferred_element_type=jnp.float32)


def _weight_prep(Ws, Ws_int, Wz, Wz_int):
    full = lambda s: pl.BlockSpec(s, lambda: tuple(0 for _ in s))
    return pl.pallas_call(
        _wprep_body,
        in_specs=[full((KCH, D, D)), full((D, D)), full((KCH, D, D)), full((D, D))],
        out_specs=full((KCH, D, 2 * D)),
        out_shape=jax.ShapeDtypeStruct((KCH, D, 2 * D), jnp.float32),
    )(Ws, Ws_int, Wz, Wz_int)


def _main_body(srow_ref, stile_ref, sflag_ref,
               da_blk, meta_blk, pool_blk, trow_blk,
               w23_ref, wp_ref, wpf_ref, out_ref, psum, tm2):
    c = pl.program_id(0)
    flags = sflag_ref[c]
    seg0 = stile_ref[c] * SG

    @pl.when(flags & 1 == 1)
    def _():
        psum[...] = jnp.zeros((SG, D), jnp.float32)
        tm2[...] = jnp.zeros((SG, 1), jnp.int32)

    @pl.when(flags & 4 == 4)
    def _():
        da = da_blk[...]
        d0 = jnp.where(meta_blk[:, 0:1] >= 0, da[:, 0, :], 0.0)
        d1 = jnp.where(meta_blk[:, 1:2] >= 0, da[:, 1, :], 0.0)
        d2 = jnp.where(meta_blk[:, 2:3] >= 0, da[:, 2, :], 0.0)

        mm = lambda a, w: lax.dot_general(a, w, (((1,), (0,)), ((), ())),
                                          preferred_element_type=jnp.float32)
        y23 = mm(d0, w23_ref[0]) + mm(d1, w23_ref[1]) + mm(d2, w23_ref[2])
        y1 = lax.dot_general(d0, wp_ref[...], (((1,), (1,)), ((), ())),
                             preferred_element_type=jnp.float32)

        tc = meta_blk[:, 3:4]
        tmp = d0
        tmp = jnp.where(tc == 3, y23[:, D:2 * D], tmp)
        tmp = jnp.where(tc == 2, y23[:, 0:D], tmp)
        tmp = jnp.where(tc == 1, y1, tmp)

        rel = pool_blk[...] - seg0
        ohT = lax.broadcasted_iota(jnp.int32, (SG, TB), 0) == rel[None, :]
        psum[...] += lax.dot_general(ohT.astype(jnp.float32), tmp,
                                     (((1,), (0,)), ((), ())),
                                     preferred_element_type=jnp.float32)
        tm2[...] = jnp.maximum(
            tm2[...],
            jnp.max(jnp.where(ohT, trow_blk[...][None, :], 0),
                    axis=1, keepdims=True))

    @pl.when(flags & 2 == 2)
    def _():
        p = psum[...]
        tm = tm2[...]
        act = jnp.where(p > 0, p, jnp.exp(p) - 1.0)
        act = jnp.where(tm != 0, act, p)
        fin = lax.dot_general(act, wpf_ref[...], (((1,), (1,)), ((), ())),
                              preferred_element_type=jnp.float32)
        out_ref[...] = jnp.where(tm == 1, fin, act)


@functools.lru_cache(maxsize=None)
def _main_call(m_pad, n_g, c_max):
    grid_spec = pltpu.PrefetchScalarGridSpec(
        num_scalar_prefetch=3,
        grid=(c_max,),
        in_specs=[
            pl.BlockSpec((TB, KCH, D), lambda c, sr, st, sf: (sr[c], 0, 0)),
            pl.BlockSpec((TB, 4), lambda c, sr, st, sf: (sr[c], 0)),
            pl.BlockSpec((TB,), lambda c, sr, st, sf: (sr[c],)),
            pl.BlockSpec((TB,), lambda c, sr, st, sf: (sr[c],)),
            pl.BlockSpec((KCH, D, 2 * D), lambda c, sr, st, sf: (0, 0, 0)),
            pl.BlockSpec((D, D), lambda c, sr, st, sf: (0, 0)),
            pl.BlockSpec((D, D), lambda c, sr, st, sf: (0, 0)),
        ],
        out_specs=pl.BlockSpec((SG, D), lambda c, sr, st, sf: (st[c], 0)),
        scratch_shapes=[
            pltpu.VMEM((SG, D), jnp.float32),
            pltpu.VMEM((SG, 1), jnp.int32),
        ],
    )
    return pl.pallas_call(
        _main_body,
        grid_spec=grid_spec,
        out_shape=jax.ShapeDtypeStruct((G_OUT, D), jnp.float32),
    )


def kernel(x, initial_map, order_matrix, type_mask, pooling,
           Ws, Ws_int, Wz, Wz_int, Wp, Wpf):
    n, d = x.shape
    k_ch, m = order_matrix.shape
    m_in = initial_map.shape[0]
    nch_sc = -(-m // (NW * CB))
    if nch_sc % 2:
        nch_sc += 1
    m_pad = NW * CB * nch_sc
    n_g = -(-G_OUT // SG)
    c_max = m // TB + n_g

    imap = initial_map.astype(jnp.int32)
    om = order_matrix.astype(jnp.int32)
    om_t = jnp.pad(om.T, ((0, m_pad - m), (0, 0)), constant_values=-1)
    om_flat = om_t.reshape(m_pad * KCH)
    om_pad = jnp.pad(om, ((0, 0), (0, m_pad - m)), constant_values=-1)
    type_pad = jnp.pad(type_mask.astype(jnp.int32), (0, m_pad - m))
    meta = jnp.concatenate([om_pad, type_pad[None, :]], axis=0).T
    pool_i = pooling.astype(jnp.int32)
    pool_pad = jnp.pad(pool_i, (0, m_pad - m), constant_values=jnp.int32(2 ** 30))

    row_start = jnp.searchsorted(
        pool_i, (jnp.arange(n_g + 1) * SG).astype(jnp.int32)).astype(jnp.int32)
    rs_al = (row_start[:-1] // TB) * TB
    nch = (row_start[1:] - rs_al + TB - 1) // TB
    coff = jnp.concatenate([jnp.zeros((1,), jnp.int32),
                            jnp.cumsum(nch, dtype=jnp.int32)])
    c_idx = jnp.arange(c_max, dtype=jnp.int32)
    tile_of = jnp.clip(
        jnp.searchsorted(coff[1:], c_idx, side="right"), 0, n_g - 1
    ).astype(jnp.int32)
    j_in = c_idx - coff[tile_of]
    real = c_idx < coff[n_g]
    srow = jnp.where(real, rs_al[tile_of] // TB + j_in, m_pad // TB - 1)
    srow = srow.astype(jnp.int32)
    sflag = (jnp.where(real & (j_in == 0), 1, 0)
             | jnp.where(real & (c_idx == coff[tile_of] + nch[tile_of] - 1), 2, 0)
             | jnp.where(real, 4, 0)).astype(jnp.int32)

    da_flat = _sc_gather(n, d, m_in, m_pad, nch_sc)(x, imap, om_flat)
    da = da_flat.reshape(m_pad, KCH, d)
    w23 = _weight_prep(Ws, Ws_int, Wz, Wz_int)
    return _main_call(m_pad, n_g, c_max)(
        srow, tile_of, sflag, da, meta, pool_pad, type_pad, w23, Wp, Wpf)
